# baseline (speedup 1.0000x reference)
import functools

import jax
import jax.numpy as jnp
from jax import lax
from jax.experimental import pallas as pl
from jax.experimental.pallas import tpu as pltpu
from jax.experimental.pallas import tpu_sc as plsc

NUM_WORKERS = 32
CHUNK = 128
SUBS = 16


def _sc_mesh():
    return plsc.VectorSubcoreMesh(core_axis_name="c", subcore_axis_name="s")


def _sc_aggregate(y, idx2, zeros_d):
    NP, D = zeros_d.shape
    NCH = idx2.shape[0] // NUM_WORKERS
    G = 8
    NGRP = NCH // G
    RPS = NP // SUBS
    out_sds = jax.ShapeDtypeStruct((NP, D), jnp.float32)

    @functools.partial(
        pl.kernel,
        out_type=(out_sds, out_sds),
        mesh=_sc_mesh(),
        scratch_types=[
            pltpu.VMEM((G, 2, CHUNK), jnp.int32),
            pltpu.VMEM((CHUNK, D), jnp.float32),
            pltpu.VMEM((CHUNK, D), jnp.float32),
            pltpu.VMEM_SHARED((NP, D), jnp.float32),
            pltpu.SemaphoreType.DMA,
            pltpu.SemaphoreType.DMA,
        ],
    )
    def sc_agg(y_hbm, i_hbm, z_hbm, oa_hbm, ob_hbm,
               idxg, rows_a, rows_b, acc, sga, sgb):
        c = lax.axis_index("c")
        s = lax.axis_index("s")
        w = s * 2 + c
        rows = (rows_a, rows_b)
        sem_g = (sga, sgb)
        pltpu.sync_copy(z_hbm.at[pl.ds(s * RPS, RPS)], acc.at[pl.ds(s * RPS, RPS)])
        plsc.subcore_barrier()

        @pl.loop(0, NGRP)
        def _(g):
            pltpu.sync_copy(i_hbm.at[pl.ds(w * NCH + g * G, G)], idxg)
            descs = [None, None]
            descs[0] = pltpu.async_copy(
                y_hbm.at[idxg.at[0].at[0]], rows[0], sem_g[0])
            for j in range(G):
                p = j & 1
                if j + 1 < G:
                    descs[1 - p] = pltpu.async_copy(
                        y_hbm.at[idxg.at[j + 1].at[0]], rows[1 - p], sem_g[1 - p])
                descs[p].wait()
                pltpu.sync_copy(rows[p], acc.at[idxg.at[j].at[1]], add=True)

        plsc.subcore_barrier()

        @pl.when(c == 0)
        def _():
            pltpu.sync_copy(acc.at[pl.ds(s * RPS, RPS)], oa_hbm.at[pl.ds(s * RPS, RPS)])

        @pl.when(c == 1)
        def _():
            pltpu.sync_copy(acc.at[pl.ds(s * RPS, RPS)], ob_hbm.at[pl.ds(s * RPS, RPS)])

    return sc_agg(y, idx2, zeros_d)


_DOT_KW = dict(preferred_element_type=jnp.float32, precision=lax.Precision.HIGHEST)


def _dis_block(da_ref, db_ref):
    return lax.rsqrt(da_ref[:, :1] + db_ref[:, :1] + 1.0)


def _tc_scale_matmul(xp, W, dpa, dpb):
    NP, D = xp.shape
    R = 512

    def body(x_ref, w_ref, da_ref, db_ref, o_ref):
        dis = _dis_block(da_ref, db_ref)
        o_ref[...] = jnp.dot(x_ref[...], w_ref[...], **_DOT_KW) * dis

    return pl.pallas_call(
        body,
        grid=(NP // R,),
        in_specs=[
            pl.BlockSpec((R, D), lambda i: (i, 0)),
            pl.BlockSpec((D, D), lambda i: (0, 0)),
            pl.BlockSpec((R, D), lambda i: (i, 0)),
            pl.BlockSpec((R, D), lambda i: (i, 0)),
        ],
        out_specs=pl.BlockSpec((R, D), lambda i: (i, 0)),
        out_shape=jax.ShapeDtypeStruct((NP, D), jnp.float32),
    )(xp, W, dpa, dpb)


def _tc_layer(pa, pb, y, dpa, dpb, b, W):
    NP, D = y.shape
    R = 512

    def body(pa_ref, pb_ref, y_ref, da_ref, db_ref, b_ref, w_ref, o_ref):
        dis = _dis_block(da_ref, db_ref)
        h = jnp.tanh(dis * (pa_ref[...] + pb_ref[...] + y_ref[...]) + b_ref[...])
        o_ref[...] = jnp.dot(h, w_ref[...], **_DOT_KW) * dis

    return pl.pallas_call(
        body,
        grid=(NP // R,),
        in_specs=[
            pl.BlockSpec((R, D), lambda i: (i, 0)),
            pl.BlockSpec((R, D), lambda i: (i, 0)),
            pl.BlockSpec((R, D), lambda i: (i, 0)),
            pl.BlockSpec((R, D), lambda i: (i, 0)),
            pl.BlockSpec((R, D), lambda i: (i, 0)),
            pl.BlockSpec((1, D), lambda i: (0, 0)),
            pl.BlockSpec((D, D), lambda i: (0, 0)),
        ],
        out_specs=pl.BlockSpec((R, D), lambda i: (i, 0)),
        out_shape=jax.ShapeDtypeStruct((NP, D), jnp.float32),
    )(pa, pb, y, dpa, dpb, b, W)


def _tc_head(pa, pb, y, dpa, dpb, b, W3p, b3p):
    NP, D = y.shape
    R = 512

    def body(pa_ref, pb_ref, y_ref, da_ref, db_ref, b_ref, w_ref, b3_ref,
             h_ref, lg_ref):
        dis = _dis_block(da_ref, db_ref)
        h = jnp.tanh(dis * (pa_ref[...] + pb_ref[...] + y_ref[...]) + b_ref[...])
        h_ref[...] = h
        lg_ref[...] = jnp.dot(h, w_ref[...], **_DOT_KW) + b3_ref[...]

    return pl.pallas_call(
        body,
        grid=(NP // R,),
        in_specs=[
            pl.BlockSpec((R, D), lambda i: (i, 0)),
            pl.BlockSpec((R, D), lambda i: (i, 0)),
            pl.BlockSpec((R, D), lambda i: (i, 0)),
            pl.BlockSpec((R, D), lambda i: (i, 0)),
            pl.BlockSpec((R, D), lambda i: (i, 0)),
            pl.BlockSpec((1, D), lambda i: (0, 0)),
            pl.BlockSpec((D, D), lambda i: (0, 0)),
            pl.BlockSpec((1, D), lambda i: (0, 0)),
        ],
        out_specs=[
            pl.BlockSpec((R, D), lambda i: (i, 0)),
            pl.BlockSpec((R, D), lambda i: (i, 0)),
        ],
        out_shape=[
            jax.ShapeDtypeStruct((NP, D), jnp.float32),
            jax.ShapeDtypeStruct((NP, D), jnp.float32),
        ],
    )(pa, pb, y, dpa, dpb, b, W3p, b3p)


def kernel(x, edge_index, W1, b1, W2, b2, W3, b3):
    N, D = x.shape
    E = edge_index.shape[1]
    NP = -(-N // 512) * 512
    NCH = -(-E // (NUM_WORKERS * CHUNK))
    NCH = -(-NCH // 8) * 8
    EPAD = NCH * NUM_WORKERS * CHUNK
    DOUT = W3.shape[1]

    src = edge_index[0].astype(jnp.int32)
    dst = edge_index[1].astype(jnp.int32)
    pad = jnp.full((EPAD - E,), N, jnp.int32)
    src2 = jnp.concatenate([src, pad]).reshape(EPAD // CHUNK, CHUNK)
    dst2 = jnp.concatenate([dst, pad]).reshape(EPAD // CHUNK, CHUNK)
    idx2 = jnp.stack([src2, dst2], axis=1)
    didx2 = jnp.stack([jnp.zeros_like(dst2), dst2], axis=1)
    xp = jnp.pad(x, ((0, NP - N), (0, 0)))
    zeros_d = jnp.zeros((NP, D), jnp.float32)
    W3p = jnp.pad(W3, ((0, 0), (0, D - DOUT)))
    b3p = jnp.pad(b3, (0, D - DOUT)).reshape(1, D)
    b1r = b1.reshape(1, D)
    b2r = b2.reshape(1, D)

    ones_tab = jnp.ones((NP, D), jnp.float32)
    dpa, dpb = _sc_aggregate(ones_tab, didx2, zeros_d)
    y1 = _tc_scale_matmul(xp, W1, dpa, dpb)
    p1a, p1b = _sc_aggregate(y1, idx2, zeros_d)
    y2 = _tc_layer(p1a, p1b, y1, dpa, dpb, b1r, W2)
    p2a, p2b = _sc_aggregate(y2, idx2, zeros_d)
    h2f, lgf = _tc_head(p2a, p2b, y2, dpa, dpb, b2r, W3p, b3p)
    return h2f[:N], lgf[:N, :DOUT]


# speedup vs baseline: 9.5901x; 9.5901x over previous
import functools

import jax
import jax.numpy as jnp
from jax import lax
from jax.experimental import pallas as pl
from jax.experimental.pallas import tpu as pltpu
from jax.experimental.pallas import tpu_sc as plsc

NUM_WORKERS = 32
CHUNK = 128
SUBS = 16


def _sc_mesh():
    return plsc.VectorSubcoreMesh(core_axis_name="c", subcore_axis_name="s")


def _sc_aggregate(y, idx2, zeros_d):
    NP, D = zeros_d.shape
    NCH = idx2.shape[0] // NUM_WORKERS
    G = 8
    NGRP = NCH // G
    RPS = NP // SUBS
    out_sds = jax.ShapeDtypeStruct((NP, D), jnp.float32)

    @functools.partial(
        pl.kernel,
        out_type=(out_sds, out_sds),
        mesh=_sc_mesh(),
        scratch_types=[
            pltpu.VMEM((G, 2, CHUNK), jnp.int32),
            pltpu.VMEM((CHUNK, D), jnp.float32),
            pltpu.VMEM((CHUNK, D), jnp.float32),
            pltpu.VMEM_SHARED((NP, D), jnp.float32),
            pltpu.SemaphoreType.DMA,
            pltpu.SemaphoreType.DMA,
        ],
    )
    def sc_agg(y_hbm, i_hbm, z_hbm, oa_hbm, ob_hbm,
               idxg, rows_a, rows_b, acc, sga, sgb):
        c = lax.axis_index("c")
        s = lax.axis_index("s")
        w = s * 2 + c
        rows = (rows_a, rows_b)
        sem_g = (sga, sgb)
        pltpu.sync_copy(z_hbm.at[pl.ds(s * RPS, RPS)], acc.at[pl.ds(s * RPS, RPS)])
        plsc.subcore_barrier()

        @pl.loop(0, NGRP)
        def _(g):
            pltpu.sync_copy(i_hbm.at[pl.ds(w * NCH + g * G, G)], idxg)
            descs = [None, None]
            descs[0] = pltpu.async_copy(
                y_hbm.at[idxg.at[0].at[0]], rows[0], sem_g[0])
            for j in range(G):
                p = j & 1
                if j + 1 < G:
                    descs[1 - p] = pltpu.async_copy(
                        y_hbm.at[idxg.at[j + 1].at[0]], rows[1 - p], sem_g[1 - p])
                descs[p].wait()
                pltpu.sync_copy(rows[p], acc.at[idxg.at[j].at[1]], add=True)

        plsc.subcore_barrier()

        @pl.when(c == 0)
        def _():
            pltpu.sync_copy(acc.at[pl.ds(s * RPS, RPS)], oa_hbm.at[pl.ds(s * RPS, RPS)])

        @pl.when(c == 1)
        def _():
            pltpu.sync_copy(acc.at[pl.ds(s * RPS, RPS)], ob_hbm.at[pl.ds(s * RPS, RPS)])

    return sc_agg(y, idx2, zeros_d)


_DOT_KW = dict(preferred_element_type=jnp.float32, precision=lax.Precision.HIGHEST)


def _dis_block(da_ref, db_ref):
    return lax.rsqrt(da_ref[:, :1] + db_ref[:, :1] + 1.0)


def _tc_scale_matmul(xp, W, dpa, dpb):
    NP, D = xp.shape
    R = 512

    def body(x_ref, w_ref, da_ref, db_ref, o_ref):
        dis = _dis_block(da_ref, db_ref)
        o_ref[...] = jnp.dot(x_ref[...], w_ref[...], **_DOT_KW) * dis

    return pl.pallas_call(
        body,
        grid=(NP // R,),
        in_specs=[
            pl.BlockSpec((R, D), lambda i: (i, 0)),
            pl.BlockSpec((D, D), lambda i: (0, 0)),
            pl.BlockSpec((R, D), lambda i: (i, 0)),
            pl.BlockSpec((R, D), lambda i: (i, 0)),
        ],
        out_specs=pl.BlockSpec((R, D), lambda i: (i, 0)),
        out_shape=jax.ShapeDtypeStruct((NP, D), jnp.float32),
    )(xp, W, dpa, dpb)


def _tc_layer(pa, pb, y, dpa, dpb, b, W):
    NP, D = y.shape
    R = 512

    def body(pa_ref, pb_ref, y_ref, da_ref, db_ref, b_ref, w_ref, o_ref):
        dis = _dis_block(da_ref, db_ref)
        h = jnp.tanh(dis * (pa_ref[...] + pb_ref[...] + y_ref[...]) + b_ref[...])
        o_ref[...] = jnp.dot(h, w_ref[...], **_DOT_KW) * dis

    return pl.pallas_call(
        body,
        grid=(NP // R,),
        in_specs=[
            pl.BlockSpec((R, D), lambda i: (i, 0)),
            pl.BlockSpec((R, D), lambda i: (i, 0)),
            pl.BlockSpec((R, D), lambda i: (i, 0)),
            pl.BlockSpec((R, D), lambda i: (i, 0)),
            pl.BlockSpec((R, D), lambda i: (i, 0)),
            pl.BlockSpec((1, D), lambda i: (0, 0)),
            pl.BlockSpec((D, D), lambda i: (0, 0)),
        ],
        out_specs=pl.BlockSpec((R, D), lambda i: (i, 0)),
        out_shape=jax.ShapeDtypeStruct((NP, D), jnp.float32),
    )(pa, pb, y, dpa, dpb, b, W)


def _tc_head(pa, pb, y, dpa, dpb, b, W3p, b3p):
    NP, D = y.shape
    R = 512

    def body(pa_ref, pb_ref, y_ref, da_ref, db_ref, b_ref, w_ref, b3_ref,
             h_ref, lg_ref):
        dis = _dis_block(da_ref, db_ref)
        h = jnp.tanh(dis * (pa_ref[...] + pb_ref[...] + y_ref[...]) + b_ref[...])
        h_ref[...] = h
        lg_ref[...] = jnp.dot(h, w_ref[...], **_DOT_KW) + b3_ref[...]

    return pl.pallas_call(
        body,
        grid=(NP // R,),
        in_specs=[
            pl.BlockSpec((R, D), lambda i: (i, 0)),
            pl.BlockSpec((R, D), lambda i: (i, 0)),
            pl.BlockSpec((R, D), lambda i: (i, 0)),
            pl.BlockSpec((R, D), lambda i: (i, 0)),
            pl.BlockSpec((R, D), lambda i: (i, 0)),
            pl.BlockSpec((1, D), lambda i: (0, 0)),
            pl.BlockSpec((D, D), lambda i: (0, 0)),
            pl.BlockSpec((1, D), lambda i: (0, 0)),
        ],
        out_specs=[
            pl.BlockSpec((R, D), lambda i: (i, 0)),
            pl.BlockSpec((R, D), lambda i: (i, 0)),
        ],
        out_shape=[
            jax.ShapeDtypeStruct((NP, D), jnp.float32),
            jax.ShapeDtypeStruct((NP, D), jnp.float32),
        ],
    )(pa, pb, y, dpa, dpb, b, W3p, b3p)


def kernel(x, edge_index, W1, b1, W2, b2, W3, b3):
    N, D = x.shape
    E = edge_index.shape[1]
    NP = -(-N // 512) * 512
    NCH = -(-E // (NUM_WORKERS * CHUNK))
    NCH = -(-NCH // 8) * 8
    EPAD = NCH * NUM_WORKERS * CHUNK
    DOUT = W3.shape[1]

    src = edge_index[0].astype(jnp.int32)
    dst = edge_index[1].astype(jnp.int32)
    pad = jnp.full((EPAD - E,), N, jnp.int32)
    src2 = jnp.concatenate([src, pad]).reshape(EPAD // CHUNK, CHUNK)
    dst2 = jnp.concatenate([dst, pad]).reshape(EPAD // CHUNK, CHUNK)
    idx2 = jnp.stack([src2, dst2], axis=1)
    didx2 = jnp.stack([dst2, dst2], axis=1)
    xp = jnp.pad(x, ((0, NP - N), (0, 0)))
    zeros_d = jnp.zeros((NP, D), jnp.float32)
    W3p = jnp.pad(W3, ((0, 0), (0, D - DOUT)))
    b3p = jnp.pad(b3, (0, D - DOUT)).reshape(1, D)
    b1r = b1.reshape(1, D)
    b2r = b2.reshape(1, D)

    ones_tab = jnp.ones((NP, D), jnp.float32)
    dpa, dpb = _sc_aggregate(ones_tab, didx2, zeros_d)
    y1 = _tc_scale_matmul(xp, W1, dpa, dpb)
    p1a, p1b = _sc_aggregate(y1, idx2, zeros_d)
    y2 = _tc_layer(p1a, p1b, y1, dpa, dpb, b1r, W2)
    p2a, p2b = _sc_aggregate(y2, idx2, zeros_d)
    h2f, lgf = _tc_head(p2a, p2b, y2, dpa, dpb, b2r, W3p, b3p)
    return h2f[:N], lgf[:N, :DOUT]


# speedup vs baseline: 15.1744x; 1.5823x over previous
import dataclasses
import functools

import jax
import jax.numpy as jnp
from jax import lax
from jax.experimental import pallas as pl
from jax.experimental.pallas import tpu as pltpu
from jax.experimental.pallas import tpu_sc as plsc

NUM_WORKERS = 32
CHUNK = 128
SUBS = 16
LANES = 16


def _sc_mesh():
    return plsc.VectorSubcoreMesh(core_axis_name="c", subcore_axis_name="s")


def _sc_params():
    cp = pltpu.CompilerParams()
    if "needs_layout_passes" in pltpu.CompilerParams.__dataclass_fields__:
        cp = dataclasses.replace(cp, needs_layout_passes=False)
    return cp


def _sc_aggregate(y, srcp, dstp, zeros_d):
    NP, D = zeros_d.shape
    NCH = srcp.shape[0] // (NUM_WORKERS * CHUNK)
    RPS = NP // SUBS
    out_sds = jax.ShapeDtypeStruct((NP, D), jnp.float32)

    @functools.partial(
        pl.kernel,
        out_type=(out_sds, out_sds),
        mesh=_sc_mesh(),
        scratch_types=[
            pltpu.VMEM((CHUNK,), jnp.int32),
            pltpu.VMEM((CHUNK,), jnp.int32),
            pltpu.VMEM((CHUNK, D), jnp.float32),
            pltpu.VMEM_SHARED((NP, D), jnp.float32),
            pltpu.SemaphoreType.DMA,
        ],
    )
    def sc_agg(y_hbm, src_hbm, dst_hbm, z_hbm, oa_hbm, ob_hbm,
               src_v, dst_v, rows_v, acc, sem):
        c = lax.axis_index("c")
        s = lax.axis_index("s")
        w = s * 2 + c
        pltpu.sync_copy(z_hbm.at[pl.ds(s * RPS, RPS)], acc.at[pl.ds(s * RPS, RPS)])
        plsc.subcore_barrier()

        @pl.loop(0, NCH)
        def _(k):
            base = (w * NCH + k) * CHUNK
            pltpu.sync_copy(src_hbm.at[pl.ds(base, CHUNK)], src_v)
            pltpu.sync_copy(dst_hbm.at[pl.ds(base, CHUNK)], dst_v)
            pltpu.async_copy(y_hbm.at[src_v], rows_v, sem).wait()
            pltpu.sync_copy(rows_v, acc.at[dst_v], add=True)

        plsc.subcore_barrier()

        @pl.when(c == 0)
        def _():
            pltpu.sync_copy(acc.at[pl.ds(s * RPS, RPS)], oa_hbm.at[pl.ds(s * RPS, RPS)])

        @pl.when(c == 1)
        def _():
            pltpu.sync_copy(acc.at[pl.ds(s * RPS, RPS)], ob_hbm.at[pl.ds(s * RPS, RPS)])

    return sc_agg(y, srcp, dstp, zeros_d)


def _sc_degree(dstp, NP):
    EW = dstp.shape[0] // NUM_WORKERS
    RPS = NP // SUBS
    out_sds = jax.ShapeDtypeStruct((NP, LANES), jnp.float32)

    @functools.partial(
        pl.kernel,
        out_type=(out_sds, out_sds),
        mesh=_sc_mesh(),
        scratch_types=[
            pltpu.VMEM((EW,), jnp.int32),
            pltpu.VMEM((NP,), jnp.float32),
            pltpu.VMEM((RPS,), jnp.float32),
            pltpu.VMEM((RPS, LANES), jnp.float32),
            pltpu.VMEM_SHARED((SUBS, NP), jnp.float32),
        ],
        compiler_params=_sc_params(),
    )
    def sc_deg(dst_hbm, oa_hbm, ob_hbm, dst_all, hist, tsum, tout, stage):
        c = lax.axis_index("c")
        s = lax.axis_index("s")
        w = s * 2 + c
        zeros = jnp.zeros((LANES,), jnp.float32)
        ones = jnp.ones((LANES,), jnp.float32)

        @pl.loop(0, NP // LANES)
        def _(i):
            hist[pl.ds(i * LANES, LANES)] = zeros

        pltpu.sync_copy(dst_hbm.at[pl.ds(w * EW, EW)], dst_all)

        @pl.loop(0, EW // LANES)
        def _(i):
            idx = dst_all[pl.ds(i * LANES, LANES)]
            plsc.addupdate_scatter(hist, [idx], ones)

        pltpu.sync_copy(hist, stage.at[s])
        plsc.subcore_barrier()

        pltpu.sync_copy(stage.at[0, pl.ds(s * RPS, RPS)], tsum)
        for t in range(1, SUBS):
            pltpu.sync_copy(stage.at[t, pl.ds(s * RPS, RPS)], hist.at[pl.ds(0, RPS)])

            @pl.loop(0, RPS // LANES)
            def _(i):
                sl = pl.ds(i * LANES, LANES)
                tsum[sl] = tsum[sl] + hist[sl]

        @pl.loop(0, RPS // LANES)
        def _(i):
            for j in range(LANES):
                r = i * LANES + j
                val = plsc.load_gather(tsum, [jnp.full((LANES,), r, jnp.int32)])
                tout[r] = val

        @pl.when(c == 0)
        def _():
            pltpu.sync_copy(tout, oa_hbm.at[pl.ds(s * RPS, RPS)])

        @pl.when(c == 1)
        def _():
            pltpu.sync_copy(tout, ob_hbm.at[pl.ds(s * RPS, RPS)])

    return sc_deg(dstp)


_DOT_KW = dict(preferred_element_type=jnp.float32, precision=lax.Precision.HIGHEST)


def _dis_block(da_ref, db_ref):
    return lax.rsqrt(da_ref[:, :1] + db_ref[:, :1] + 1.0)


def _tc_scale_matmul(xp, W, dpa, dpb):
    NP, D = xp.shape
    R = 512

    def body(x_ref, w_ref, da_ref, db_ref, o_ref):
        dis = _dis_block(da_ref, db_ref)
        o_ref[...] = jnp.dot(x_ref[...], w_ref[...], **_DOT_KW) * dis

    return pl.pallas_call(
        body,
        grid=(NP // R,),
        in_specs=[
            pl.BlockSpec((R, D), lambda i: (i, 0)),
            pl.BlockSpec((D, D), lambda i: (0, 0)),
            pl.BlockSpec((R, LANES), lambda i: (i, 0)),
            pl.BlockSpec((R, LANES), lambda i: (i, 0)),
        ],
        out_specs=pl.BlockSpec((R, D), lambda i: (i, 0)),
        out_shape=jax.ShapeDtypeStruct((NP, D), jnp.float32),
    )(xp, W, dpa, dpb)


def _tc_layer(pa, pb, y, dpa, dpb, b, W):
    NP, D = y.shape
    R = 512

    def body(pa_ref, pb_ref, y_ref, da_ref, db_ref, b_ref, w_ref, o_ref):
        dis = _dis_block(da_ref, db_ref)
        h = jnp.tanh(dis * (pa_ref[...] + pb_ref[...] + y_ref[...]) + b_ref[...])
        o_ref[...] = jnp.dot(h, w_ref[...], **_DOT_KW) * dis

    return pl.pallas_call(
        body,
        grid=(NP // R,),
        in_specs=[
            pl.BlockSpec((R, D), lambda i: (i, 0)),
            pl.BlockSpec((R, D), lambda i: (i, 0)),
            pl.BlockSpec((R, D), lambda i: (i, 0)),
            pl.BlockSpec((R, LANES), lambda i: (i, 0)),
            pl.BlockSpec((R, LANES), lambda i: (i, 0)),
            pl.BlockSpec((1, D), lambda i: (0, 0)),
            pl.BlockSpec((D, D), lambda i: (0, 0)),
        ],
        out_specs=pl.BlockSpec((R, D), lambda i: (i, 0)),
        out_shape=jax.ShapeDtypeStruct((NP, D), jnp.float32),
    )(pa, pb, y, dpa, dpb, b, W)


def _tc_head(pa, pb, y, dpa, dpb, b, W3p, b3p):
    NP, D = y.shape
    R = 512

    def body(pa_ref, pb_ref, y_ref, da_ref, db_ref, b_ref, w_ref, b3_ref,
             h_ref, lg_ref):
        dis = _dis_block(da_ref, db_ref)
        h = jnp.tanh(dis * (pa_ref[...] + pb_ref[...] + y_ref[...]) + b_ref[...])
        h_ref[...] = h
        lg_ref[...] = jnp.dot(h, w_ref[...], **_DOT_KW) + b3_ref[...]

    return pl.pallas_call(
        body,
        grid=(NP // R,),
        in_specs=[
            pl.BlockSpec((R, D), lambda i: (i, 0)),
            pl.BlockSpec((R, D), lambda i: (i, 0)),
            pl.BlockSpec((R, D), lambda i: (i, 0)),
            pl.BlockSpec((R, LANES), lambda i: (i, 0)),
            pl.BlockSpec((R, LANES), lambda i: (i, 0)),
            pl.BlockSpec((1, D), lambda i: (0, 0)),
            pl.BlockSpec((D, D), lambda i: (0, 0)),
            pl.BlockSpec((1, D), lambda i: (0, 0)),
        ],
        out_specs=[
            pl.BlockSpec((R, D), lambda i: (i, 0)),
            pl.BlockSpec((R, D), lambda i: (i, 0)),
        ],
        out_shape=[
            jax.ShapeDtypeStruct((NP, D), jnp.float32),
            jax.ShapeDtypeStruct((NP, D), jnp.float32),
        ],
    )(pa, pb, y, dpa, dpb, b, W3p, b3p)


def kernel(x, edge_index, W1, b1, W2, b2, W3, b3):
    N, D = x.shape
    E = edge_index.shape[1]
    NP = -(-N // 512) * 512
    NCH = -(-E // (NUM_WORKERS * CHUNK))
    EPAD = NCH * NUM_WORKERS * CHUNK
    DOUT = W3.shape[1]

    src = edge_index[0].astype(jnp.int32)
    dst = edge_index[1].astype(jnp.int32)
    pad = jnp.full((EPAD - E,), N, jnp.int32)
    srcp = jnp.concatenate([src, pad])
    dstp = jnp.concatenate([dst, pad])
    xp = jnp.pad(x, ((0, NP - N), (0, 0)))
    zeros_d = jnp.zeros((NP, D), jnp.float32)
    W3p = jnp.pad(W3, ((0, 0), (0, D - DOUT)))
    b3p = jnp.pad(b3, (0, D - DOUT)).reshape(1, D)
    b1r = b1.reshape(1, D)
    b2r = b2.reshape(1, D)

    dpa, dpb = _sc_degree(dstp, NP)
    y1 = _tc_scale_matmul(xp, W1, dpa, dpb)
    p1a, p1b = _sc_aggregate(y1, srcp, dstp, zeros_d)
    y2 = _tc_layer(p1a, p1b, y1, dpa, dpb, b1r, W2)
    p2a, p2b = _sc_aggregate(y2, srcp, dstp, zeros_d)
    h2f, lgf = _tc_head(p2a, p2b, y2, dpa, dpb, b2r, W3p, b3p)
    return h2f[:N], lgf[:N, :DOUT]
